# baseline (speedup 1.0000x reference)
import functools

import jax
import jax.numpy as jnp
from jax import lax
from jax.experimental import pallas as pl
from jax.experimental.pallas import tpu as pltpu
from jax.experimental.pallas import tpu_sc as plsc

N_ROWS = 100000
D = 64
B_ROWS = 16384

NC = 2
NS = 16
L = 16

CHUNKS_PER_CORE = 4
CHUNK = 12504
LAST_CHUNK = N_ROWS - (NC * CHUNKS_PER_CORE - 1) * CHUNK
TRASH = 64

B_PER_TILE = B_ROWS // NS
A_PER_TILE = 784
A_LAST = CHUNK - (NS - 1) * A_PER_TILE
A_LAST_FINAL = LAST_CHUNK - (NS - 1) * A_PER_TILE

IDX_W = 128
N_IDX_ROWS = B_PER_TILE // IDX_W

_mesh = plsc.VectorSubcoreMesh(core_axis_name="c", subcore_axis_name="s")


@functools.partial(
    pl.kernel,
    mesh=_mesh,
    out_type=jax.ShapeDtypeStruct((N_ROWS, D), jnp.float32),
    scratch_types=[
        pltpu.VMEM((B_PER_TILE, D), jnp.float32),
        pltpu.VMEM((B_PER_TILE,), jnp.int32),
        pltpu.VMEM((N_IDX_ROWS, IDX_W), jnp.int32),
        pltpu.VMEM_SHARED((CHUNK + TRASH, D), jnp.float32),
    ],
    compiler_params=pltpu.CompilerParams(use_tc_tiling_on_sc=False),
)
def _scatter_add(idx_hbm, a_hbm, b_hbm, out_hbm, b_v, idx_v, sidx_v, chunk_sh):
    c = lax.axis_index("c")
    s = lax.axis_index("s")

    pltpu.sync_copy(b_hbm.at[pl.ds(s * B_PER_TILE, B_PER_TILE)], b_v)
    pltpu.sync_copy(idx_hbm.at[pl.ds(s * B_PER_TILE, B_PER_TILE)], idx_v)

    for k in range(CHUNKS_PER_CORE):
        base = (c * CHUNKS_PER_CORE + k) * CHUNK
        is_final = k == CHUNKS_PER_CORE - 1

        @pl.when(s < NS - 1)
        def _():
            pltpu.sync_copy(
                a_hbm.at[pl.ds(base + s * A_PER_TILE, A_PER_TILE)],
                chunk_sh.at[pl.ds(s * A_PER_TILE, A_PER_TILE)],
            )

        @pl.when((s == NS - 1) & (jnp.bool_(not is_final) | (c == 0)))
        def _():
            pltpu.sync_copy(
                a_hbm.at[pl.ds(base + (NS - 1) * A_PER_TILE, A_LAST)],
                chunk_sh.at[pl.ds((NS - 1) * A_PER_TILE, A_LAST)],
            )

        if is_final:
            @pl.when((s == NS - 1) & (c == 1))
            def _():
                pltpu.sync_copy(
                    a_hbm.at[pl.ds(base + (NS - 1) * A_PER_TILE, A_LAST_FINAL)],
                    chunk_sh.at[pl.ds((NS - 1) * A_PER_TILE, A_LAST_FINAL)],
                )

        lanes = lax.iota(jnp.int32, L)
        for i in range(B_PER_TILE // L):
            v = idx_v[pl.ds(i * L, L)]
            local = v - base
            in_chunk = (local >= 0) & (local < CHUNK)
            trash = lanes + jnp.int32(CHUNK + (i % (TRASH // L)) * L)
            sidx_v[i // (IDX_W // L), pl.ds((i % (IDX_W // L)) * L, L)] = (
                jnp.where(in_chunk, local, trash)
            )

        plsc.subcore_barrier()

        for j in range(N_IDX_ROWS):
            pltpu.sync_copy(
                b_v.at[pl.ds(j * IDX_W, IDX_W)],
                chunk_sh.at[sidx_v.at[j]],
                add=True,
            )

        plsc.subcore_barrier()

        @pl.when(s < NS - 1)
        def _():
            pltpu.sync_copy(
                chunk_sh.at[pl.ds(s * A_PER_TILE, A_PER_TILE)],
                out_hbm.at[pl.ds(base + s * A_PER_TILE, A_PER_TILE)],
            )

        @pl.when((s == NS - 1) & (jnp.bool_(not is_final) | (c == 0)))
        def _():
            pltpu.sync_copy(
                chunk_sh.at[pl.ds((NS - 1) * A_PER_TILE, A_LAST)],
                out_hbm.at[pl.ds(base + (NS - 1) * A_PER_TILE, A_LAST)],
            )

        if is_final:
            @pl.when((s == NS - 1) & (c == 1))
            def _():
                pltpu.sync_copy(
                    chunk_sh.at[pl.ds((NS - 1) * A_PER_TILE, A_LAST_FINAL)],
                    out_hbm.at[pl.ds(base + (NS - 1) * A_PER_TILE, A_LAST_FINAL)],
                )

        if k != CHUNKS_PER_CORE - 1:
            plsc.subcore_barrier()


def kernel(index, A, B):
    return _scatter_add(index.astype(jnp.int32), A, B)
